# speedup vs baseline: 5.5371x; 5.5371x over previous
import functools

import jax
import jax.numpy as jnp
from jax import lax
from jax.experimental import pallas as pl
from jax.experimental.pallas import tpu as pltpu
from jax.experimental.pallas import tpu_sc as plsc

LN_EPS = 1e-12

_GATHER_WINDOW = 256
_TC_BLK = 1024


def _sc_gather_rows(table, idx, n, d):
    idx2 = idx.reshape(1, n)
    mesh = plsc.VectorSubcoreMesh(core_axis_name="core",
                                  subcore_axis_name="subcore")

    @functools.partial(
        pl.kernel,
        out_type=jax.ShapeDtypeStruct((n, d), table.dtype),
        mesh=mesh,
    )
    def gather_kernel(tab_hbm, i_hbm, o_hbm):
        def body(i_vmem, o_vmem):
            pltpu.sync_copy(tab_hbm.at[i_vmem.at[0]], o_vmem)

        pltpu.emit_pipeline(
            body,
            grid=(n // _GATHER_WINDOW,),
            in_specs=[pl.BlockSpec((1, _GATHER_WINDOW),
                                   index_map=lambda i: (0, i))],
            out_specs=[pl.BlockSpec((_GATHER_WINDOW, d),
                                    index_map=lambda i: (i, 0))],
            core_axis_name=("core", "subcore"),
            dimension_semantics=(pltpu.PARALLEL,),
        )(i_hbm, o_hbm)

    return gather_kernel(table, idx2)


def _tc_embed_ln(we, pid, tt, pos_table, type_table, gamma, beta):
    n, d = we.shape
    max_pos = pos_table.shape[0]
    nb = n // _TC_BLK

    def body(we_ref, pid_ref, tt_ref, ptab_ref, ttab_ref, g_ref, b_ref,
             o_ref):
        w = we_ref[...]
        p = pid_ref[...]
        oh = (p == lax.broadcasted_iota(jnp.int32, (_TC_BLK, max_pos), 1))
        pe = jnp.dot(oh.astype(jnp.float32), ptab_ref[...],
                     preferred_element_type=jnp.float32)
        t0 = ttab_ref[0:1, :]
        t1 = ttab_ref[1:2, :]
        te = t0 + tt_ref[...].astype(jnp.float32) * (t1 - t0)
        emb = w + pe + te
        mean = jnp.mean(emb, axis=1, keepdims=True)
        cen = emb - mean
        var = jnp.mean(cen * cen, axis=1, keepdims=True)
        o_ref[...] = cen * lax.rsqrt(var + LN_EPS) * g_ref[...] + b_ref[...]

    return pl.pallas_call(
        body,
        grid=(nb,),
        in_specs=[
            pl.BlockSpec((_TC_BLK, d), lambda i: (i, 0)),
            pl.BlockSpec((_TC_BLK, 1), lambda i: (i, 0)),
            pl.BlockSpec((_TC_BLK, 1), lambda i: (i, 0)),
            pl.BlockSpec((max_pos, d), lambda i: (0, 0)),
            pl.BlockSpec(type_table.shape, lambda i: (0, 0)),
            pl.BlockSpec((1, d), lambda i: (0, 0)),
            pl.BlockSpec((1, d), lambda i: (0, 0)),
        ],
        out_specs=pl.BlockSpec((_TC_BLK, d), lambda i: (i, 0)),
        out_shape=jax.ShapeDtypeStruct((n, d), jnp.float32),
    )(we, pid, tt, pos_table, type_table, gamma, beta)


def kernel(input_ids, position_ids, token_type_ids, inputs_embeds,
           word_table, pos_table, type_table, ln_gamma, ln_beta):
    b, l = position_ids.shape
    d = word_table.shape[1]
    n = b * l

    ids = input_ids[:, :, 0].astype(jnp.int32).reshape(n)
    we = _sc_gather_rows(word_table, ids, n, d)

    pid = position_ids.astype(jnp.int32).reshape(n, 1)
    tt = token_type_ids.astype(jnp.int32).reshape(n, 1)
    out = _tc_embed_ln(we, pid, tt, pos_table, type_table,
                       ln_gamma.reshape(1, d), ln_beta.reshape(1, d))
    return out.reshape(b, l, d)
